# baseline (device time: 48503 ns/iter reference)
import jax
import jax.numpy as jnp
from jax import lax
from jax.experimental import pallas as pl
from jax.experimental.pallas import tpu as pltpu

N_DEV = 4


def kernel(x, router_W, route_idx, expert_W):
    m, d = x.shape
    e_loc, _, h = expert_W.shape
    n_exp = N_DEV * e_loc

    def body(x_ref, rw_ref, idx_ref, ew_ref, out_ref,
             comm_ref, send_sems, recv_sems):
        my = lax.axis_index("i")
        left = jnp.mod(my - 1, N_DEV)
        right = jnp.mod(my + 1, N_DEV)

        barrier_sem = pltpu.get_barrier_semaphore()
        for nbr in (left, right):
            pl.semaphore_signal(
                barrier_sem, inc=1,
                device_id=(nbr,), device_id_type=pl.DeviceIdType.MESH,
            )
        pl.semaphore_wait(barrier_sem, 2)

        xv = x_ref[:, :]
        scores = jnp.dot(xv, rw_ref[:, :], preferred_element_type=jnp.float32)
        p = jnp.exp(scores - jnp.max(scores, axis=-1, keepdims=True))
        p = p / jnp.sum(p, axis=-1, keepdims=True)
        iota = lax.broadcasted_iota(jnp.int32, (m, n_exp), 1)
        mask = jnp.logical_or(
            idx_ref[:, 0:1] == iota, idx_ref[:, 1:2] == iota
        ).astype(jnp.float32)
        pm = p * mask
        gates = pm / jnp.sum(pm, axis=-1, keepdims=True)

        def contrib(origin, w_ref):
            acc = None
            for j in range(e_loc):
                sel = (iota == origin * e_loc + j).astype(jnp.float32)
                g = jnp.sum(gates * sel, axis=-1, keepdims=True)
                t = jnp.dot(xv * g, w_ref[j],
                            preferred_element_type=jnp.float32)
                acc = t if acc is None else acc + t
            return acc

        out_ref[:, :] = contrib(my, ew_ref)

        for hop in range(N_DEV - 1):
            src = ew_ref if hop == 0 else comm_ref.at[hop - 1]
            rdma = pltpu.make_async_remote_copy(
                src_ref=src,
                dst_ref=comm_ref.at[hop],
                send_sem=send_sems.at[hop],
                recv_sem=recv_sems.at[hop],
                device_id=(right,),
                device_id_type=pl.DeviceIdType.MESH,
            )
            rdma.start()
            rdma.wait()
            origin = jnp.mod(my - hop - 1, N_DEV)
            out_ref[:, :] += contrib(origin, comm_ref.at[hop])

    return pl.pallas_call(
        body,
        out_shape=jax.ShapeDtypeStruct((m, h), jnp.float32),
        in_specs=[
            pl.BlockSpec(memory_space=pltpu.VMEM),
            pl.BlockSpec(memory_space=pltpu.VMEM),
            pl.BlockSpec(memory_space=pltpu.VMEM),
            pl.BlockSpec(memory_space=pltpu.VMEM),
        ],
        out_specs=pl.BlockSpec(memory_space=pltpu.VMEM),
        scratch_shapes=[
            pltpu.VMEM((N_DEV - 1, e_loc, d, h), jnp.float32),
            pltpu.SemaphoreType.DMA((N_DEV - 1,)),
            pltpu.SemaphoreType.DMA((N_DEV - 1,)),
        ],
        compiler_params=pltpu.CompilerParams(collective_id=0),
    )(x, router_W, route_idx, expert_W)


# device time: 30011 ns/iter; 1.6162x vs baseline; 1.6162x over previous
import jax
import jax.numpy as jnp
from jax import lax
from jax.experimental import pallas as pl
from jax.experimental.pallas import tpu as pltpu

N_DEV = 4


def kernel(x, router_W, route_idx, expert_W):
    m, d = x.shape
    e_loc, _, h = expert_W.shape
    n_exp = N_DEV * e_loc

    def body(x_ref, rw_ref, idx_ref, ew_ref, out_ref,
             bw_ref, comm_ref, send_sems, recv_sems):
        my = lax.axis_index("i")
        left = jnp.mod(my - 1, N_DEV)
        right = jnp.mod(my + 1, N_DEV)

        barrier_sem = pltpu.get_barrier_semaphore()
        for nbr in (left, right):
            pl.semaphore_signal(
                barrier_sem, inc=1,
                device_id=(nbr,), device_id_type=pl.DeviceIdType.MESH,
            )
        pl.semaphore_wait(barrier_sem, 2)

        bw_ref[:, :, :] = ew_ref[:, :, :].astype(jnp.bfloat16)

        def hop_rdma(hop):
            return pltpu.make_async_remote_copy(
                src_ref=bw_ref if hop == 0 else comm_ref.at[hop - 1],
                dst_ref=comm_ref.at[hop],
                send_sem=send_sems.at[hop],
                recv_sem=recv_sems.at[hop],
                device_id=(right,),
                device_id_type=pl.DeviceIdType.MESH,
            )

        rdmas = [hop_rdma(0)]
        rdmas[0].start()

        xv = x_ref[:, :]
        scores = jnp.dot(xv, rw_ref[:, :], preferred_element_type=jnp.float32)
        p = jnp.exp(scores - jnp.max(scores, axis=-1, keepdims=True))
        p = p / jnp.sum(p, axis=-1, keepdims=True)
        iota = lax.broadcasted_iota(jnp.int32, (m, n_exp), 1)
        mask = jnp.logical_or(
            idx_ref[:, 0:1] == iota, idx_ref[:, 1:2] == iota
        ).astype(jnp.float32)
        pm = p * mask
        gates = pm / jnp.sum(pm, axis=-1, keepdims=True)

        def contrib(origin, w_ref):
            acc = None
            for j in range(e_loc):
                sel = (iota == origin * e_loc + j).astype(jnp.float32)
                g = jnp.sum(gates * sel, axis=-1, keepdims=True)
                t = jnp.dot((xv * g).astype(jnp.bfloat16), w_ref[j],
                            preferred_element_type=jnp.float32)
                acc = t if acc is None else acc + t
            return acc

        out_ref[:, :] = contrib(my, bw_ref)

        for hop in range(N_DEV - 1):
            rdmas[hop].wait_recv()
            if hop < N_DEV - 2:
                rdmas.append(hop_rdma(hop + 1))
                rdmas[hop + 1].start()
            origin = jnp.mod(my - hop - 1, N_DEV)
            out_ref[:, :] += contrib(origin, comm_ref.at[hop])

        for r in rdmas:
            r.wait_send()

    return pl.pallas_call(
        body,
        out_shape=jax.ShapeDtypeStruct((m, h), jnp.float32),
        in_specs=[
            pl.BlockSpec(memory_space=pltpu.VMEM),
            pl.BlockSpec(memory_space=pltpu.VMEM),
            pl.BlockSpec(memory_space=pltpu.VMEM),
            pl.BlockSpec(memory_space=pltpu.VMEM),
        ],
        out_specs=pl.BlockSpec(memory_space=pltpu.VMEM),
        scratch_shapes=[
            pltpu.VMEM((e_loc, d, h), jnp.bfloat16),
            pltpu.VMEM((N_DEV - 1, e_loc, d, h), jnp.bfloat16),
            pltpu.SemaphoreType.DMA((N_DEV - 1,)),
            pltpu.SemaphoreType.DMA((N_DEV - 1,)),
        ],
        compiler_params=pltpu.CompilerParams(collective_id=0),
    )(x, router_W, route_idx, expert_W)


# device time: 22308 ns/iter; 2.1742x vs baseline; 1.3453x over previous
import jax
import jax.numpy as jnp
from jax import lax
from jax.experimental import pallas as pl
from jax.experimental.pallas import tpu as pltpu

N_DEV = 4


def kernel(x, router_W, route_idx, expert_W):
    m, d = x.shape
    e_loc, _, h = expert_W.shape
    n_exp = N_DEV * e_loc

    def body(x_ref, rw_ref, idx_ref, ew_ref, out_ref,
             bw_ref, comm_ref, send_sems, recv_sems):
        my = lax.axis_index("i")
        left = jnp.mod(my - 1, N_DEV)
        right = jnp.mod(my + 1, N_DEV)

        barrier_sem = pltpu.get_barrier_semaphore()
        for nbr in (left, right):
            pl.semaphore_signal(
                barrier_sem, inc=1,
                device_id=(nbr,), device_id_type=pl.DeviceIdType.MESH,
            )
        pl.semaphore_wait(barrier_sem, 2)

        bw_ref[:, :, :] = ew_ref[:, :, :].astype(jnp.bfloat16)

        def flow(src, dst_slot, sem, dev):
            return pltpu.make_async_remote_copy(
                src_ref=src,
                dst_ref=comm_ref.at[dst_slot],
                send_sem=send_sems.at[sem],
                recv_sem=recv_sems.at[sem],
                device_id=(dev,),
                device_id_type=pl.DeviceIdType.MESH,
            )

        r_left = flow(bw_ref, 1, 0, left)
        r_right = flow(bw_ref, 0, 1, right)
        r_left.start()
        r_right.start()

        xv = x_ref[:, :]
        scores = jnp.dot(xv, rw_ref[:, :], preferred_element_type=jnp.float32)
        p = jnp.exp(scores - jnp.max(scores, axis=-1, keepdims=True))
        p = p / jnp.sum(p, axis=-1, keepdims=True)
        iota = lax.broadcasted_iota(jnp.int32, (m, n_exp), 1)
        mask = jnp.logical_or(
            idx_ref[:, 0:1] == iota, idx_ref[:, 1:2] == iota
        ).astype(jnp.float32)
        pm = p * mask
        gates = pm / jnp.sum(pm, axis=-1, keepdims=True)

        def contrib(origin, w_ref):
            acc = None
            for j in range(e_loc):
                sel = (iota == origin * e_loc + j).astype(jnp.float32)
                g = jnp.sum(gates * sel, axis=-1, keepdims=True)
                t = jnp.dot((xv * g).astype(jnp.bfloat16), w_ref[j],
                            preferred_element_type=jnp.float32)
                acc = t if acc is None else acc + t
            return acc

        out_ref[:, :] = contrib(my, bw_ref)

        r_left.wait_recv()
        r_fwd = flow(comm_ref.at[1], 2, 2, left)
        r_fwd.start()
        out_ref[:, :] += contrib(jnp.mod(my + 1, N_DEV), comm_ref.at[1])

        r_right.wait_recv()
        out_ref[:, :] += contrib(jnp.mod(my - 1, N_DEV), comm_ref.at[0])

        r_fwd.wait_recv()
        out_ref[:, :] += contrib(jnp.mod(my + 2, N_DEV), comm_ref.at[2])

        for r in (r_left, r_right, r_fwd):
            r.wait_send()

    return pl.pallas_call(
        body,
        out_shape=jax.ShapeDtypeStruct((m, h), jnp.float32),
        in_specs=[
            pl.BlockSpec(memory_space=pltpu.VMEM),
            pl.BlockSpec(memory_space=pltpu.VMEM),
            pl.BlockSpec(memory_space=pltpu.VMEM),
            pl.BlockSpec(memory_space=pltpu.VMEM),
        ],
        out_specs=pl.BlockSpec(memory_space=pltpu.VMEM),
        scratch_shapes=[
            pltpu.VMEM((e_loc, d, h), jnp.bfloat16),
            pltpu.VMEM((N_DEV - 1, e_loc, d, h), jnp.bfloat16),
            pltpu.SemaphoreType.DMA((N_DEV - 1,)),
            pltpu.SemaphoreType.DMA((N_DEV - 1,)),
        ],
        compiler_params=pltpu.CompilerParams(collective_id=0),
    )(x, router_W, route_idx, expert_W)


# device time: 20950 ns/iter; 2.3152x vs baseline; 1.0648x over previous
import jax
import jax.numpy as jnp
from jax import lax
from jax.experimental import pallas as pl
from jax.experimental.pallas import tpu as pltpu

N_DEV = 4


def kernel(x, router_W, route_idx, expert_W):
    m, d = x.shape
    e_loc, _, h = expert_W.shape
    n_exp = N_DEV * e_loc

    def body(x_ref, rw_ref, idx_ref, ew_ref, out_ref,
             bw_ref, comm_ref, send_sems, recv_sems):
        my = lax.axis_index("i")
        left = jnp.mod(my - 1, N_DEV)
        right = jnp.mod(my + 1, N_DEV)

        barrier_sem = pltpu.get_barrier_semaphore()
        for nbr in (left, right):
            pl.semaphore_signal(
                barrier_sem, inc=1,
                device_id=(nbr,), device_id_type=pl.DeviceIdType.MESH,
            )
        pl.semaphore_wait(barrier_sem, 2)

        bw_ref[:, :, :] = ew_ref[:, :, :].astype(jnp.bfloat16)

        def flow(src, dst, sem, dev):
            return pltpu.make_async_remote_copy(
                src_ref=src,
                dst_ref=dst,
                send_sem=send_sems.at[sem],
                recv_sem=recv_sems.at[sem],
                device_id=(dev,),
                device_id_type=pl.DeviceIdType.MESH,
            )

        r_right = flow(bw_ref, comm_ref.at[0], 0, right)
        r_l = [flow(bw_ref.at[j], comm_ref.at[1, j], 1 + j, left)
               for j in range(e_loc)]
        for r in r_l:
            r.start()
        r_right.start()

        xv = x_ref[:, :]
        scores = jnp.dot(xv, rw_ref[:, :], preferred_element_type=jnp.float32)
        p = jnp.exp(scores - jnp.max(scores, axis=-1, keepdims=True))
        p = p / jnp.sum(p, axis=-1, keepdims=True)
        iota = lax.broadcasted_iota(jnp.int32, (m, n_exp), 1)
        mask = jnp.logical_or(
            idx_ref[:, 0:1] == iota, idx_ref[:, 1:2] == iota
        ).astype(jnp.float32)
        pm = p * mask
        gates = pm / jnp.sum(pm, axis=-1, keepdims=True)

        def contrib_one(e_id, w2d_ref):
            sel = (iota == e_id).astype(jnp.float32)
            g = jnp.sum(gates * sel, axis=-1, keepdims=True)
            return jnp.dot((xv * g).astype(jnp.bfloat16), w2d_ref[:, :],
                           preferred_element_type=jnp.float32)

        def contrib(origin, w_ref):
            acc = None
            for j in range(e_loc):
                t = contrib_one(origin * e_loc + j, w_ref.at[j])
                acc = t if acc is None else acc + t
            return acc

        out_ref[:, :] = contrib(my, bw_ref)

        r_fwd = []
        for j in range(e_loc):
            r_l[j].wait_recv()
            r_fwd.append(flow(comm_ref.at[1, j], comm_ref.at[2, j],
                              1 + e_loc + j, left))
            r_fwd[j].start()
            out_ref[:, :] += contrib_one(
                jnp.mod(my + 1, N_DEV) * e_loc + j, comm_ref.at[1, j])

        r_right.wait_recv()
        out_ref[:, :] += contrib(jnp.mod(my - 1, N_DEV), comm_ref.at[0])

        for j in range(e_loc):
            r_fwd[j].wait_recv()
            out_ref[:, :] += contrib_one(
                jnp.mod(my + 2, N_DEV) * e_loc + j, comm_ref.at[2, j])

        for r in [r_right] + r_l + r_fwd:
            r.wait_send()

    return pl.pallas_call(
        body,
        out_shape=jax.ShapeDtypeStruct((m, h), jnp.float32),
        in_specs=[
            pl.BlockSpec(memory_space=pltpu.VMEM),
            pl.BlockSpec(memory_space=pltpu.VMEM),
            pl.BlockSpec(memory_space=pltpu.VMEM),
            pl.BlockSpec(memory_space=pltpu.VMEM),
        ],
        out_specs=pl.BlockSpec(memory_space=pltpu.VMEM),
        scratch_shapes=[
            pltpu.VMEM((e_loc, d, h), jnp.bfloat16),
            pltpu.VMEM((N_DEV - 1, e_loc, d, h), jnp.bfloat16),
            pltpu.SemaphoreType.DMA((1 + 2 * e_loc,)),
            pltpu.SemaphoreType.DMA((1 + 2 * e_loc,)),
        ],
        compiler_params=pltpu.CompilerParams(collective_id=0),
    )(x, router_W, route_idx, expert_W)


# device time: 19725 ns/iter; 2.4590x vs baseline; 1.0621x over previous
import jax
import jax.numpy as jnp
from jax import lax
from jax.experimental import pallas as pl
from jax.experimental.pallas import tpu as pltpu

N_DEV = 4


def kernel(x, router_W, route_idx, expert_W):
    m, d = x.shape
    e_loc, _, h = expert_W.shape
    n_exp = N_DEV * e_loc

    def body(x_ref, rw_ref, idx_ref, ew_ref, out_ref,
             bw_ref, comm_ref, send_sems, recv_sems):
        my = lax.axis_index("i")
        left = jnp.mod(my - 1, N_DEV)
        right = jnp.mod(my + 1, N_DEV)

        barrier_sem = pltpu.get_barrier_semaphore()
        for nbr in (left, right):
            pl.semaphore_signal(
                barrier_sem, inc=1,
                device_id=(nbr,), device_id_type=pl.DeviceIdType.MESH,
            )
        bw_ref[:, :, :] = ew_ref[:, :, :].astype(jnp.bfloat16)
        pl.semaphore_wait(barrier_sem, 2)

        def flow(src, dst, sem, dev):
            return pltpu.make_async_remote_copy(
                src_ref=src,
                dst_ref=dst,
                send_sem=send_sems.at[sem],
                recv_sem=recv_sems.at[sem],
                device_id=(dev,),
                device_id_type=pl.DeviceIdType.MESH,
            )

        r_right = flow(bw_ref, comm_ref.at[0], 0, right)
        r_l = [flow(bw_ref.at[j], comm_ref.at[1, j], 1 + j, left)
               for j in range(e_loc)]
        for r in r_l:
            r.start()
        r_right.start()

        xv = x_ref[:, :]
        scores = jnp.dot(xv, rw_ref[:, :], preferred_element_type=jnp.float32)
        p = jnp.exp(scores - jnp.max(scores, axis=-1, keepdims=True))
        p = p / jnp.sum(p, axis=-1, keepdims=True)
        iota = lax.broadcasted_iota(jnp.int32, (m, n_exp), 1)
        mask = jnp.logical_or(
            idx_ref[:, 0:1] == iota, idx_ref[:, 1:2] == iota
        ).astype(jnp.float32)
        pm = p * mask
        gates = pm / jnp.sum(pm, axis=-1, keepdims=True)

        def contrib_one(e_id, w2d_ref):
            sel = (iota == e_id).astype(jnp.float32)
            g = jnp.sum(gates * sel, axis=-1, keepdims=True)
            return jnp.dot((xv * g).astype(jnp.bfloat16), w2d_ref[:, :],
                           preferred_element_type=jnp.float32)

        def contrib(origin, w_ref):
            acc = None
            for j in range(e_loc):
                t = contrib_one(origin * e_loc + j, w_ref.at[j])
                acc = t if acc is None else acc + t
            return acc

        out_ref[:, :] = contrib(my, bw_ref)

        r_fwd = []
        for j in range(e_loc):
            r_l[j].wait_recv()
            r_fwd.append(flow(comm_ref.at[1, j], comm_ref.at[2, j],
                              1 + e_loc + j, left))
            r_fwd[j].start()
            out_ref[:, :] += contrib_one(
                jnp.mod(my + 1, N_DEV) * e_loc + j, comm_ref.at[1, j])

        r_right.wait_recv()
        out_ref[:, :] += contrib(jnp.mod(my - 1, N_DEV), comm_ref.at[0])

        for j in range(e_loc):
            r_fwd[j].wait_recv()
            out_ref[:, :] += contrib_one(
                jnp.mod(my + 2, N_DEV) * e_loc + j, comm_ref.at[2, j])

        for r in [r_right] + r_l + r_fwd:
            r.wait_send()

    return pl.pallas_call(
        body,
        out_shape=jax.ShapeDtypeStruct((m, h), jnp.float32),
        in_specs=[
            pl.BlockSpec(memory_space=pltpu.VMEM),
            pl.BlockSpec(memory_space=pltpu.VMEM),
            pl.BlockSpec(memory_space=pltpu.VMEM),
            pl.BlockSpec(memory_space=pltpu.VMEM),
        ],
        out_specs=pl.BlockSpec(memory_space=pltpu.VMEM),
        scratch_shapes=[
            pltpu.VMEM((e_loc, d, h), jnp.bfloat16),
            pltpu.VMEM((N_DEV - 1, e_loc, d, h), jnp.bfloat16),
            pltpu.SemaphoreType.DMA((1 + 2 * e_loc,)),
            pltpu.SemaphoreType.DMA((1 + 2 * e_loc,)),
        ],
        compiler_params=pltpu.CompilerParams(collective_id=0),
    )(x, router_W, route_idx, expert_W)
